# baseline (speedup 1.0000x reference)
import jax
import jax.numpy as jnp
from jax import lax
from jax.experimental import pallas as pl
from jax.experimental.pallas import tpu as pltpu

P, K, D = 1024, 8192, 256
KT = 2048
NSTEP = K // KT


def _vq_body(a_ref, b_ref, ids_ref, out_ref, best_val):
    p = pl.program_id(0)
    j = pl.program_id(1)
    b = b_ref[...]

    @pl.when(p == 0)
    def _argmin_phase():
        a = a_ref[...]
        a2 = jnp.sum(a * a, axis=1, keepdims=True)
        b2 = jnp.sum(b * b, axis=1)[None, :]
        mm = lax.dot_general(a, b, (((1,), (1,)), ((), ())),
                             preferred_element_type=jnp.float32)
        d2 = (a2 + b2) - 2.0 * mm
        dist = jnp.sqrt(jnp.maximum(d2, 0.0))
        local_min = jnp.min(dist, axis=1, keepdims=True)
        col = lax.broadcasted_iota(jnp.int32, (P, KT), 1) + j * KT
        local_idx = jnp.min(jnp.where(dist == local_min, col, K),
                            axis=1, keepdims=True)

        @pl.when(j == 0)
        def _():
            best_val[...] = local_min
            ids_ref[...] = local_idx

        @pl.when(j > 0)
        def _():
            better = local_min < best_val[...]
            best_val[...] = jnp.where(better, local_min, best_val[...])
            ids_ref[...] = jnp.where(better, local_idx, ids_ref[...])

    @pl.when(p == 1)
    def _lookup_phase():
        col = lax.broadcasted_iota(jnp.int32, (P, KT), 1) + j * KT
        onehot = (col == ids_ref[...]).astype(jnp.float32)
        part = lax.dot_general(onehot, b, (((1,), (0,)), ((), ())),
                               preferred_element_type=jnp.float32)

        @pl.when(j == 0)
        def _():
            out_ref[...] = part

        @pl.when(j > 0)
        def _():
            out_ref[...] = out_ref[...] + part


def kernel(prompt_embs, clip_embs):
    ids2d, out_embs = pl.pallas_call(
        _vq_body,
        grid=(2, NSTEP),
        in_specs=[
            pl.BlockSpec((P, D), lambda p, j: (0, 0)),
            pl.BlockSpec((KT, D), lambda p, j: (j, 0)),
        ],
        out_specs=[
            pl.BlockSpec((P, 1), lambda p, j: (0, 0)),
            pl.BlockSpec((P, D), lambda p, j: (0, 0)),
        ],
        out_shape=[
            jax.ShapeDtypeStruct((P, 1), jnp.int32),
            jax.ShapeDtypeStruct((P, D), jnp.float32),
        ],
        scratch_shapes=[pltpu.VMEM((P, 1), jnp.float32)],
    )(prompt_embs, clip_embs)
    return (out_embs, ids2d.reshape(P))


# speedup vs baseline: 1.4539x; 1.4539x over previous
import jax
import jax.numpy as jnp
from jax import lax
from jax.experimental import pallas as pl
from jax.experimental.pallas import tpu as pltpu

P, K, D = 1024, 8192, 256
KT = 4096
NSTEP = K // KT


def _vq_body(a_ref, b_ref, ids_ref, out_ref, best_val):
    p = pl.program_id(0)
    j = pl.program_id(1)
    b = b_ref[...]

    @pl.when(p == 0)
    def _argmin_phase():
        a = a_ref[...]
        a2 = jnp.sum(a * a, axis=1, keepdims=True)
        b2 = jnp.sum(b * b, axis=1)[None, :]
        mm = lax.dot_general(a, b, (((1,), (1,)), ((), ())),
                             preferred_element_type=jnp.float32)
        d2 = (a2 + b2) - 2.0 * mm
        local_min = jnp.min(d2, axis=1, keepdims=True)
        col = lax.broadcasted_iota(jnp.int32, (P, KT), 1) + j * KT
        local_idx = jnp.min(jnp.where(d2 == local_min, col, K),
                            axis=1, keepdims=True)

        @pl.when(j == 0)
        def _():
            best_val[...] = local_min
            ids_ref[...] = local_idx

        @pl.when(j > 0)
        def _():
            better = local_min < best_val[...]
            best_val[...] = jnp.where(better, local_min, best_val[...])
            ids_ref[...] = jnp.where(better, local_idx, ids_ref[...])

    @pl.when(p == 1)
    def _lookup_phase():
        col = lax.broadcasted_iota(jnp.int32, (P, KT), 1) + j * KT
        onehot = (col == ids_ref[...]).astype(jnp.float32)
        part = lax.dot_general(onehot, b, (((1,), (0,)), ((), ())),
                               preferred_element_type=jnp.float32)

        @pl.when(j == 0)
        def _():
            out_ref[...] = part

        @pl.when(j > 0)
        def _():
            out_ref[...] = out_ref[...] + part


def kernel(prompt_embs, clip_embs):
    ids2d, out_embs = pl.pallas_call(
        _vq_body,
        grid=(2, NSTEP),
        in_specs=[
            pl.BlockSpec((P, D), lambda p, j: (0, 0)),
            pl.BlockSpec((KT, D), lambda p, j: (j, 0)),
        ],
        out_specs=[
            pl.BlockSpec((P, 1), lambda p, j: (0, 0)),
            pl.BlockSpec((P, D), lambda p, j: (0, 0)),
        ],
        out_shape=[
            jax.ShapeDtypeStruct((P, 1), jnp.int32),
            jax.ShapeDtypeStruct((P, D), jnp.float32),
        ],
        scratch_shapes=[pltpu.VMEM((P, 1), jnp.float32)],
    )(prompt_embs, clip_embs)
    return (out_embs, ids2d.reshape(P))


# speedup vs baseline: 1.4629x; 1.0062x over previous
import jax
import jax.numpy as jnp
from jax import lax
from jax.experimental import pallas as pl
from jax.experimental.pallas import tpu as pltpu

P, K, D = 1024, 8192, 256
KT = 8192
NSTEP = K // KT


def _vq_body(a_ref, b_ref, ids_ref, out_ref, best_val):
    p = pl.program_id(0)
    j = pl.program_id(1)
    b = b_ref[...]

    @pl.when(p == 0)
    def _argmin_phase():
        a = a_ref[...]
        a2 = jnp.sum(a * a, axis=1, keepdims=True)
        b2 = jnp.sum(b * b, axis=1)[None, :]
        mm = lax.dot_general(a, b, (((1,), (1,)), ((), ())),
                             preferred_element_type=jnp.float32)
        d2 = (a2 + b2) - 2.0 * mm
        local_min = jnp.min(d2, axis=1, keepdims=True)
        col = lax.broadcasted_iota(jnp.int32, (P, KT), 1) + j * KT
        local_idx = jnp.min(jnp.where(d2 == local_min, col, K),
                            axis=1, keepdims=True)

        @pl.when(j == 0)
        def _():
            best_val[...] = local_min
            ids_ref[...] = local_idx

        @pl.when(j > 0)
        def _():
            better = local_min < best_val[...]
            best_val[...] = jnp.where(better, local_min, best_val[...])
            ids_ref[...] = jnp.where(better, local_idx, ids_ref[...])

    @pl.when(p == 1)
    def _lookup_phase():
        col = lax.broadcasted_iota(jnp.int32, (P, KT), 1) + j * KT
        onehot = (col == ids_ref[...]).astype(jnp.float32)
        part = lax.dot_general(onehot, b, (((1,), (0,)), ((), ())),
                               preferred_element_type=jnp.float32)

        @pl.when(j == 0)
        def _():
            out_ref[...] = part

        @pl.when(j > 0)
        def _():
            out_ref[...] = out_ref[...] + part


def kernel(prompt_embs, clip_embs):
    ids2d, out_embs = pl.pallas_call(
        _vq_body,
        grid=(2, NSTEP),
        in_specs=[
            pl.BlockSpec((P, D), lambda p, j: (0, 0)),
            pl.BlockSpec((KT, D), lambda p, j: (j, 0)),
        ],
        out_specs=[
            pl.BlockSpec((P, 1), lambda p, j: (0, 0)),
            pl.BlockSpec((P, D), lambda p, j: (0, 0)),
        ],
        out_shape=[
            jax.ShapeDtypeStruct((P, 1), jnp.int32),
            jax.ShapeDtypeStruct((P, D), jnp.float32),
        ],
        scratch_shapes=[pltpu.VMEM((P, 1), jnp.float32)],
    )(prompt_embs, clip_embs)
    return (out_embs, ids2d.reshape(P))
